# baseline (speedup 1.0000x reference)
import functools

import numpy as np
import jax
import jax.numpy as jnp
from jax import lax
from jax.experimental import pallas as pl
from jax.experimental.pallas import tpu as pltpu
from jax.experimental.pallas import tpu_sc as plsc

B, C, H, W = 4, 1, 384, 384
DELTA_COS = 0.867
DELTA_Z = 1e-05
SAMPLE_RATIO = 0.2
EPS = 1e-06
DXYZ = 0.005
HW = H * W
G = int(HW * SAMPLE_RATIO)
GP = 29696
NSLICE = 16
NPW = 3 * B * GP // NSLICE
CHUNK = 7424
NCHUNK = GP // CHUNK


_U32 = np.uint32


def _tf2x32(k1, k2, x0, x1):
    def rotl(x, d):
        return ((x << _U32(d)) | (x >> _U32(32 - d))).astype(np.uint32)

    ks = [k1, k2, (k1 ^ k2 ^ _U32(0x1BD11BDA)).astype(np.uint32)]
    x = [(x0 + ks[0]).astype(np.uint32), (x1 + ks[1]).astype(np.uint32)]

    def rounds(x, rots):
        for r in rots:
            a = (x[0] + x[1]).astype(np.uint32)
            b = a ^ rotl(x[1], r)
            x = [a, b]
        return x

    r0 = (13, 15, 26, 6)
    r1 = (17, 29, 16, 24)
    x = rounds(x, r0)
    x = [(x[0] + ks[1]).astype(np.uint32), (x[1] + ks[2] + _U32(1)).astype(np.uint32)]
    x = rounds(x, r1)
    x = [(x[0] + ks[2]).astype(np.uint32), (x[1] + ks[0] + _U32(2)).astype(np.uint32)]
    x = rounds(x, r0)
    x = [(x[0] + ks[0]).astype(np.uint32), (x[1] + ks[1] + _U32(3)).astype(np.uint32)]
    x = rounds(x, r1)
    x = [(x[0] + ks[1]).astype(np.uint32), (x[1] + ks[2] + _U32(4)).astype(np.uint32)]
    x = rounds(x, r0)
    x = [(x[0] + ks[2]).astype(np.uint32), (x[1] + ks[0] + _U32(5)).astype(np.uint32)]
    return x[0], x[1]


def _fold_in(key, data):
    o0, o1 = _tf2x32(key[0], key[1], _U32(data >> 32), _U32(data & 0xFFFFFFFF))
    return np.array([o0, o1], np.uint32)


def _split2(key):
    b1, b2 = _tf2x32(key[0], key[1], np.zeros(2, np.uint32),
                     np.arange(2, dtype=np.uint32))
    return (np.array([b1[0], b2[0]], np.uint32),
            np.array([b1[1], b2[1]], np.uint32))


def _np_permutation(key, n):
    x = np.arange(n, dtype=np.int32)
    num_rounds = int(np.ceil(3 * np.log(n) / np.log(np.iinfo(np.uint32).max)))
    for _ in range(num_rounds):
        key, subkey = _split2(key)
        b1, b2 = _tf2x32(subkey[0], subkey[1], np.zeros(n, np.uint32),
                         np.arange(n, dtype=np.uint32))
        x = x[np.argsort(b1 ^ b2, kind="stable")]
    return x


def _build_pix():
    base = np.array([0, 1234], np.uint32)
    out = np.zeros((3, B, G), np.int32)
    for j in range(3):
        kj = _fold_in(base, j)
        for i in range(B):
            out[j, i] = _np_permutation(_fold_in(kj, i), HW)[:G]
    return out


_pix_np = np.zeros((3, B, GP), np.int32)
with np.errstate(over="ignore"):
    _pix_np[:, :, :G] = _build_pix()
_scidx_np = (_pix_np + (np.arange(B, dtype=np.int32) * HW)[None, :, None]).reshape(NSLICE, NPW)

_RC = ((_pix_np // W) * 512 + (_pix_np % W)).astype(np.float32)
_base_idx = _scidx_np.reshape(-1)
_SCIDX = np.stack([(2 * _base_idx).reshape(32, -1),
                   (2 * _base_idx + 1).reshape(32, -1)], axis=1)


NW = 32
NPW2 = 3 * B * GP // NW
EIGHTH = GP // 8


def _sc_gather_body(tab_hbm, idx_hbm, out_hbm, idx_t, idx_p, tbuf, pbuf,
                    sem_t, sem_p):
    c = lax.axis_index("c")
    s = lax.axis_index("s")
    w = c * 16 + s
    pltpu.sync_copy(idx_hbm.at[w, 0], idx_t)
    pltpu.sync_copy(idx_hbm.at[w, 1], idx_p)
    cp_t = pltpu.async_copy(tab_hbm.at[idx_t], tbuf, sem_t)
    cp_p = pltpu.async_copy(tab_hbm.at[idx_p], pbuf, sem_p)
    cp_t.wait()
    cp_p.wait()
    for q in range(3):
        eg = w * 3 + q
        j = eg // (8 * B)
        r = eg % (8 * B)
        b = r // 8
        ei = r % 8
        pltpu.sync_copy(tbuf.at[pl.ds(q * EIGHTH, EIGHTH)],
                        out_hbm.at[0, j, b, pl.ds(ei * EIGHTH, EIGHTH)])
        pltpu.sync_copy(pbuf.at[pl.ds(q * EIGHTH, EIGHTH)],
                        out_hbm.at[1, j, b, pl.ds(ei * EIGHTH, EIGHTH)])


@functools.lru_cache(maxsize=None)
def _get_sc_gather():
    return pl.kernel(
        _sc_gather_body,
        out_type=jax.ShapeDtypeStruct((2, 3, B, GP), jnp.float32),
        mesh=plsc.VectorSubcoreMesh(core_axis_name="c", subcore_axis_name="s"),
        scratch_types=[
            pltpu.VMEM((NPW2,), jnp.int32),
            pltpu.VMEM((NPW2,), jnp.int32),
            pltpu.VMEM((NPW2,), jnp.float32),
            pltpu.VMEM((NPW2,), jnp.float32),
            pltpu.SemaphoreType.DMA,
            pltpu.SemaphoreType.DMA,
        ],
    )


def _sub3(a, b):
    return (a[0] - b[0], a[1] - b[1], a[2] - b[2])


def _dot3(a, b):
    return a[0] * b[0] + a[1] * b[1] + a[2] * b[2]


def _cross3(a, b):
    return (a[1] * b[2] - a[2] * b[1],
            a[2] * b[0] - a[0] * b[2],
            a[0] * b[1] - a[1] * b[0])


def _tc_loss_body(gat_ref, pix_ref, intr_ref, out_ref, acc):
    pid = pl.program_id(0)

    @pl.when(pid == 0)
    def _():
        acc[0] = 0.0
        acc[1] = 0.0

    ir = intr_ref[...]
    rf = 1.0 / ir[:, 0:1]
    u0 = ir[:, 2:3]
    v0 = ir[:, 5:6]

    gat = gat_ref[...]
    rc = pix_ref[...]
    rowf = jnp.floor(rc * (1.0 / 512.0))
    colf = rc - rowf * 512.0

    gt, pr = [], []
    for k in range(3):
        cu = (colf[k] - u0) * rf
        cv = (rowf[k] - v0) * rf
        zt = gat[0, k]
        zp = gat[1, k]
        gt.append((cu * zt, cv * zt, zt))
        pr.append((cu * zp, cv * zp, zp))

    ds = (_sub3(gt[1], gt[0]), _sub3(gt[2], gt[0]), _sub3(gt[2], gt[1]))
    e = [[None] * 3 for _ in range(3)]
    for a in range(3):
        for b in range(a, 3):
            e[a][b] = _dot3(ds[a], ds[b])
            e[b][a] = e[a][b]
    qn = [jnp.sqrt(e[a][a]) for a in range(3)]

    hit = {}
    for a in range(3):
        for b in range(a, 3):
            thr = DELTA_COS * (qn[a] * qn[b] + EPS)
            hit[(a, b)] = (jnp.abs(e[a][b]) > thr).astype(jnp.int32)
    cnt = (hit[(0, 0)] + hit[(1, 1)] + hit[(2, 2)]
           + 2 * (hit[(0, 1)] + hit[(0, 2)] + hit[(1, 2)]))
    mask_cos = cnt > 3
    mask_pad = (gt[0][2] > DELTA_Z) & (gt[1][2] > DELTA_Z) & (gt[2][2] > DELTA_Z)
    mxyz = []
    for coord in range(3):
        mxyz.append((jnp.abs(ds[0][coord]) < DXYZ)
                    | (jnp.abs(ds[1][coord]) < DXYZ)
                    | (jnp.abs(ds[2][coord]) < DXYZ))
    keep = mask_pad & ~((mxyz[0] & mxyz[1] & mxyz[2]) | mask_cos)

    zc = [pr[k][2] == 0.0 for k in range(3)]
    prq = []
    for k in range(3):
        prq.append((jnp.where(zc[0], 1e-4, pr[k][0]),
                    jnp.where(zc[1], 1e-4, pr[k][1]),
                    jnp.where(zc[2], 1e-4, pr[k][2])))

    pp12 = _sub3(prq[1], prq[0])
    pp13 = _sub3(prq[2], prq[0])
    gn = _cross3(ds[0], ds[1])
    pn = _cross3(pp12, pp13)
    gnn = jnp.sqrt(_dot3(gn, gn))
    pnn = jnp.sqrt(_dot3(pn, pn))
    ign = 1.0 / (gnn + (gnn == 0.0).astype(jnp.float32) * EPS)
    ipn = 1.0 / (pnn + (pnn == 0.0).astype(jnp.float32) * EPS)
    lm = (jnp.abs(gn[0] * ign - pn[0] * ipn)
          + jnp.abs(gn[1] * ign - pn[1] * ipn)
          + jnp.abs(gn[2] * ign - pn[2] * ipn))

    gidx = pid * CHUNK + lax.broadcasted_iota(jnp.int32, (B, CHUNK), 1)
    w = (keep & (gidx < G)).astype(jnp.float32)
    acc[0] += jnp.sum(lm * w)
    acc[1] += jnp.sum(w)

    @pl.when(pid == NCHUNK - 1)
    def _():
        out_ref[0, 0] = acc[0] / (acc[1] * 3.0 + EPS)


_tc_loss = pl.pallas_call(
    _tc_loss_body,
    grid=(NCHUNK,),
    in_specs=[
        pl.BlockSpec((2, 3, B, CHUNK), lambda i: (0, 0, 0, i)),
        pl.BlockSpec((3, B, CHUNK), lambda i: (0, 0, i)),
        pl.BlockSpec((B, 9), lambda i: (0, 0)),
    ],
    out_specs=pl.BlockSpec((1, 1), lambda i: (0, 0), memory_space=pltpu.SMEM),
    out_shape=jax.ShapeDtypeStruct((1, 1), jnp.float32),
    scratch_shapes=[pltpu.SMEM((2,), jnp.float32)],
)


def kernel(prediction, target, mask, intrinsic):
    tab = jnp.stack([target.reshape(B * HW), prediction.reshape(B * HW)],
                    axis=-1).reshape(2 * B * HW)
    gat = _get_sc_gather()(tab, _SCIDX)
    out = _tc_loss(gat, _RC, intrinsic.reshape(B, 9))
    return out.reshape(())


# speedup vs baseline: 6.2053x; 6.2053x over previous
import functools

import numpy as np
import jax
import jax.numpy as jnp
from jax import lax
from jax.experimental import pallas as pl
from jax.experimental.pallas import tpu as pltpu
from jax.experimental.pallas import tpu_sc as plsc

B, C, H, W = 4, 1, 384, 384
DELTA_COS = 0.867
DELTA_Z = 1e-05
SAMPLE_RATIO = 0.2
EPS = 1e-06
DXYZ = 0.005
HW = H * W
G = int(HW * SAMPLE_RATIO)
GP = 29696
NSLICE = 16
NPW = 3 * B * GP // NSLICE
CHUNK = 7424
NCHUNK = GP // CHUNK


_U32 = np.uint32


def _tf2x32(k1, k2, x0, x1):
    def rotl(x, d):
        return ((x << _U32(d)) | (x >> _U32(32 - d))).astype(np.uint32)

    ks = [k1, k2, (k1 ^ k2 ^ _U32(0x1BD11BDA)).astype(np.uint32)]
    x = [(x0 + ks[0]).astype(np.uint32), (x1 + ks[1]).astype(np.uint32)]

    def rounds(x, rots):
        for r in rots:
            a = (x[0] + x[1]).astype(np.uint32)
            b = a ^ rotl(x[1], r)
            x = [a, b]
        return x

    r0 = (13, 15, 26, 6)
    r1 = (17, 29, 16, 24)
    x = rounds(x, r0)
    x = [(x[0] + ks[1]).astype(np.uint32), (x[1] + ks[2] + _U32(1)).astype(np.uint32)]
    x = rounds(x, r1)
    x = [(x[0] + ks[2]).astype(np.uint32), (x[1] + ks[0] + _U32(2)).astype(np.uint32)]
    x = rounds(x, r0)
    x = [(x[0] + ks[0]).astype(np.uint32), (x[1] + ks[1] + _U32(3)).astype(np.uint32)]
    x = rounds(x, r1)
    x = [(x[0] + ks[1]).astype(np.uint32), (x[1] + ks[2] + _U32(4)).astype(np.uint32)]
    x = rounds(x, r0)
    x = [(x[0] + ks[2]).astype(np.uint32), (x[1] + ks[0] + _U32(5)).astype(np.uint32)]
    return x[0], x[1]


def _fold_in(key, data):
    o0, o1 = _tf2x32(key[0], key[1], _U32(data >> 32), _U32(data & 0xFFFFFFFF))
    return np.array([o0, o1], np.uint32)


def _split2(key):
    b1, b2 = _tf2x32(key[0], key[1], np.zeros(2, np.uint32),
                     np.arange(2, dtype=np.uint32))
    return (np.array([b1[0], b2[0]], np.uint32),
            np.array([b1[1], b2[1]], np.uint32))


def _np_permutation(key, n):
    x = np.arange(n, dtype=np.int32)
    num_rounds = int(np.ceil(3 * np.log(n) / np.log(np.iinfo(np.uint32).max)))
    for _ in range(num_rounds):
        key, subkey = _split2(key)
        b1, b2 = _tf2x32(subkey[0], subkey[1], np.zeros(n, np.uint32),
                         np.arange(n, dtype=np.uint32))
        x = x[np.argsort(b1 ^ b2, kind="stable")]
    return x


def _build_pix():
    base = np.array([0, 1234], np.uint32)
    out = np.zeros((3, B, G), np.int32)
    for j in range(3):
        kj = _fold_in(base, j)
        for i in range(B):
            out[j, i] = _np_permutation(_fold_in(kj, i), HW)[:G]
    return out


_pix_np = np.zeros((3, B, GP), np.int32)
with np.errstate(over="ignore"):
    _pix_np[:, :, :G] = _build_pix()
_scidx_np = (_pix_np + (np.arange(B, dtype=np.int32) * HW)[None, :, None]).reshape(NSLICE, NPW)

_RC = ((_pix_np // W) * 512 + (_pix_np % W)).astype(np.float32)
_base_idx = _scidx_np.reshape(-1)
_SCIDX = _scidx_np


NW = 32
NPW2 = 3 * B * GP // NW
EIGHTH = GP // 8


QUARTER = GP // 4


def _sc_gather_body(targ_hbm, pred_hbm, idx_hbm, out_hbm, idx_v, rows_v, sem):
    c = lax.axis_index("c")
    s = lax.axis_index("s")
    pltpu.sync_copy(idx_hbm.at[s], idx_v)

    @pl.when(c == 0)
    def _():
        pltpu.async_copy(targ_hbm.at[idx_v], rows_v, sem).wait()

    @pl.when(c == 1)
    def _():
        pltpu.async_copy(pred_hbm.at[idx_v], rows_v, sem).wait()

    for q in range(3):
        qg = s * 3 + q
        j = qg // (4 * B)
        r = qg % (4 * B)
        b = r // 4
        qi = r % 4
        pltpu.sync_copy(rows_v.at[pl.ds(q * QUARTER, QUARTER)],
                        out_hbm.at[c, j, b, pl.ds(qi * QUARTER, QUARTER)])


@functools.lru_cache(maxsize=None)
def _get_sc_gather():
    return pl.kernel(
        _sc_gather_body,
        out_type=jax.ShapeDtypeStruct((2, 3, B, GP), jnp.float32),
        mesh=plsc.VectorSubcoreMesh(core_axis_name="c", subcore_axis_name="s"),
        scratch_types=[
            pltpu.VMEM((NPW,), jnp.int32),
            pltpu.VMEM((NPW,), jnp.float32),
            pltpu.SemaphoreType.DMA,
        ],
    )


def _sub3(a, b):
    return (a[0] - b[0], a[1] - b[1], a[2] - b[2])


def _dot3(a, b):
    return a[0] * b[0] + a[1] * b[1] + a[2] * b[2]


def _cross3(a, b):
    return (a[1] * b[2] - a[2] * b[1],
            a[2] * b[0] - a[0] * b[2],
            a[0] * b[1] - a[1] * b[0])


def _tc_loss_body(gat_ref, pix_ref, intr_ref, out_ref, acc):
    pid = pl.program_id(0)

    @pl.when(pid == 0)
    def _():
        acc[0] = 0.0
        acc[1] = 0.0

    ir = intr_ref[...]
    rf = 1.0 / ir[:, 0:1]
    u0 = ir[:, 2:3]
    v0 = ir[:, 5:6]

    gat = gat_ref[...]
    rc = pix_ref[...]
    rowf = jnp.floor(rc * (1.0 / 512.0))
    colf = rc - rowf * 512.0

    gt, pr = [], []
    for k in range(3):
        cu = (colf[k] - u0) * rf
        cv = (rowf[k] - v0) * rf
        zt = gat[0, k]
        zp = gat[1, k]
        gt.append((cu * zt, cv * zt, zt))
        pr.append((cu * zp, cv * zp, zp))

    ds = (_sub3(gt[1], gt[0]), _sub3(gt[2], gt[0]), _sub3(gt[2], gt[1]))
    e = [[None] * 3 for _ in range(3)]
    for a in range(3):
        for b in range(a, 3):
            e[a][b] = _dot3(ds[a], ds[b])
            e[b][a] = e[a][b]
    qn = [jnp.sqrt(e[a][a]) for a in range(3)]

    hit = {}
    for a in range(3):
        for b in range(a, 3):
            thr = DELTA_COS * (qn[a] * qn[b] + EPS)
            hit[(a, b)] = (jnp.abs(e[a][b]) > thr).astype(jnp.int32)
    cnt = (hit[(0, 0)] + hit[(1, 1)] + hit[(2, 2)]
           + 2 * (hit[(0, 1)] + hit[(0, 2)] + hit[(1, 2)]))
    mask_cos = cnt > 3
    mask_pad = (gt[0][2] > DELTA_Z) & (gt[1][2] > DELTA_Z) & (gt[2][2] > DELTA_Z)
    mxyz = []
    for coord in range(3):
        mxyz.append((jnp.abs(ds[0][coord]) < DXYZ)
                    | (jnp.abs(ds[1][coord]) < DXYZ)
                    | (jnp.abs(ds[2][coord]) < DXYZ))
    keep = mask_pad & ~((mxyz[0] & mxyz[1] & mxyz[2]) | mask_cos)

    zc = [pr[k][2] == 0.0 for k in range(3)]
    prq = []
    for k in range(3):
        prq.append((jnp.where(zc[0], 1e-4, pr[k][0]),
                    jnp.where(zc[1], 1e-4, pr[k][1]),
                    jnp.where(zc[2], 1e-4, pr[k][2])))

    pp12 = _sub3(prq[1], prq[0])
    pp13 = _sub3(prq[2], prq[0])
    gn = _cross3(ds[0], ds[1])
    pn = _cross3(pp12, pp13)
    gnn = jnp.sqrt(_dot3(gn, gn))
    pnn = jnp.sqrt(_dot3(pn, pn))
    ign = 1.0 / (gnn + (gnn == 0.0).astype(jnp.float32) * EPS)
    ipn = 1.0 / (pnn + (pnn == 0.0).astype(jnp.float32) * EPS)
    lm = (jnp.abs(gn[0] * ign - pn[0] * ipn)
          + jnp.abs(gn[1] * ign - pn[1] * ipn)
          + jnp.abs(gn[2] * ign - pn[2] * ipn))

    gidx = pid * CHUNK + lax.broadcasted_iota(jnp.int32, (B, CHUNK), 1)
    w = (keep & (gidx < G)).astype(jnp.float32)
    acc[0] += jnp.sum(lm * w)
    acc[1] += jnp.sum(w)

    @pl.when(pid == NCHUNK - 1)
    def _():
        out_ref[0, 0] = acc[0] / (acc[1] * 3.0 + EPS)


_tc_loss = pl.pallas_call(
    _tc_loss_body,
    grid=(NCHUNK,),
    in_specs=[
        pl.BlockSpec((2, 3, B, CHUNK), lambda i: (0, 0, 0, i)),
        pl.BlockSpec((3, B, CHUNK), lambda i: (0, 0, i)),
        pl.BlockSpec((B, 9), lambda i: (0, 0)),
    ],
    out_specs=pl.BlockSpec((1, 1), lambda i: (0, 0), memory_space=pltpu.SMEM),
    out_shape=jax.ShapeDtypeStruct((1, 1), jnp.float32),
    scratch_shapes=[pltpu.SMEM((2,), jnp.float32)],
)


def kernel(prediction, target, mask, intrinsic):
    targ_flat = target.reshape(B * HW)
    pred_flat = prediction.reshape(B * HW)
    gat = _get_sc_gather()(targ_flat, pred_flat, _SCIDX)
    out = _tc_loss(gat, _RC, intrinsic.reshape(B, 9))
    return out.reshape(())


# speedup vs baseline: 6.2686x; 1.0102x over previous
import functools

import numpy as np
import jax
import jax.numpy as jnp
from jax import lax
from jax.experimental import pallas as pl
from jax.experimental.pallas import tpu as pltpu
from jax.experimental.pallas import tpu_sc as plsc

B, C, H, W = 4, 1, 384, 384
DELTA_COS = 0.867
DELTA_Z = 1e-05
SAMPLE_RATIO = 0.2
EPS = 1e-06
DXYZ = 0.005
HW = H * W
G = int(HW * SAMPLE_RATIO)
GP = 29696
NSLICE = 16
NPW = 3 * B * GP // NSLICE
CHUNK = 7424
NCHUNK = GP // CHUNK


_U32 = np.uint32


def _tf2x32(k1, k2, x0, x1):
    def rotl(x, d):
        return ((x << _U32(d)) | (x >> _U32(32 - d))).astype(np.uint32)

    ks = [k1, k2, (k1 ^ k2 ^ _U32(0x1BD11BDA)).astype(np.uint32)]
    x = [(x0 + ks[0]).astype(np.uint32), (x1 + ks[1]).astype(np.uint32)]

    def rounds(x, rots):
        for r in rots:
            a = (x[0] + x[1]).astype(np.uint32)
            b = a ^ rotl(x[1], r)
            x = [a, b]
        return x

    r0 = (13, 15, 26, 6)
    r1 = (17, 29, 16, 24)
    x = rounds(x, r0)
    x = [(x[0] + ks[1]).astype(np.uint32), (x[1] + ks[2] + _U32(1)).astype(np.uint32)]
    x = rounds(x, r1)
    x = [(x[0] + ks[2]).astype(np.uint32), (x[1] + ks[0] + _U32(2)).astype(np.uint32)]
    x = rounds(x, r0)
    x = [(x[0] + ks[0]).astype(np.uint32), (x[1] + ks[1] + _U32(3)).astype(np.uint32)]
    x = rounds(x, r1)
    x = [(x[0] + ks[1]).astype(np.uint32), (x[1] + ks[2] + _U32(4)).astype(np.uint32)]
    x = rounds(x, r0)
    x = [(x[0] + ks[2]).astype(np.uint32), (x[1] + ks[0] + _U32(5)).astype(np.uint32)]
    return x[0], x[1]


def _fold_in(key, data):
    o0, o1 = _tf2x32(key[0], key[1], _U32(data >> 32), _U32(data & 0xFFFFFFFF))
    return np.array([o0, o1], np.uint32)


def _split2(key):
    b1, b2 = _tf2x32(key[0], key[1], np.zeros(2, np.uint32),
                     np.arange(2, dtype=np.uint32))
    return (np.array([b1[0], b2[0]], np.uint32),
            np.array([b1[1], b2[1]], np.uint32))


def _np_permutation(key, n):
    x = np.arange(n, dtype=np.int32)
    num_rounds = int(np.ceil(3 * np.log(n) / np.log(np.iinfo(np.uint32).max)))
    for _ in range(num_rounds):
        key, subkey = _split2(key)
        b1, b2 = _tf2x32(subkey[0], subkey[1], np.zeros(n, np.uint32),
                         np.arange(n, dtype=np.uint32))
        x = x[np.argsort(b1 ^ b2, kind="stable")]
    return x


def _build_pix():
    base = np.array([0, 1234], np.uint32)
    out = np.zeros((3, B, G), np.int32)
    for j in range(3):
        kj = _fold_in(base, j)
        for i in range(B):
            out[j, i] = _np_permutation(_fold_in(kj, i), HW)[:G]
    return out


_pix_np = np.zeros((3, B, GP), np.int32)
with np.errstate(over="ignore"):
    _raw_pix = _build_pix()
for _b in range(B):
    _order = np.argsort(_raw_pix[0, _b], kind="stable")
    _raw_pix[:, _b, :] = _raw_pix[:, _b, _order]
_pix_np[:, :, :G] = _raw_pix
_scidx_np = (_pix_np + (np.arange(B, dtype=np.int32) * HW)[None, :, None]).reshape(NSLICE, NPW)

_RC = ((_pix_np // W) * 512 + (_pix_np % W)).astype(np.float32)
_base_idx = _scidx_np.reshape(-1)
_SCIDX = _scidx_np


NW = 32
NPW2 = 3 * B * GP // NW
EIGHTH = GP // 8


QUARTER = GP // 4


def _sc_gather_body(targ_hbm, pred_hbm, idx_hbm, out_hbm, idx_v, rows_v, sem):
    c = lax.axis_index("c")
    s = lax.axis_index("s")
    pltpu.sync_copy(idx_hbm.at[s], idx_v)

    @pl.when(c == 0)
    def _():
        pltpu.async_copy(targ_hbm.at[idx_v], rows_v, sem).wait()

    @pl.when(c == 1)
    def _():
        pltpu.async_copy(pred_hbm.at[idx_v], rows_v, sem).wait()

    for q in range(3):
        qg = s * 3 + q
        j = qg // (4 * B)
        r = qg % (4 * B)
        b = r // 4
        qi = r % 4
        pltpu.sync_copy(rows_v.at[pl.ds(q * QUARTER, QUARTER)],
                        out_hbm.at[c, j, b, pl.ds(qi * QUARTER, QUARTER)])


@functools.lru_cache(maxsize=None)
def _get_sc_gather():
    return pl.kernel(
        _sc_gather_body,
        out_type=jax.ShapeDtypeStruct((2, 3, B, GP), jnp.float32),
        mesh=plsc.VectorSubcoreMesh(core_axis_name="c", subcore_axis_name="s"),
        scratch_types=[
            pltpu.VMEM((NPW,), jnp.int32),
            pltpu.VMEM((NPW,), jnp.float32),
            pltpu.SemaphoreType.DMA,
        ],
    )


def _sub3(a, b):
    return (a[0] - b[0], a[1] - b[1], a[2] - b[2])


def _dot3(a, b):
    return a[0] * b[0] + a[1] * b[1] + a[2] * b[2]


def _cross3(a, b):
    return (a[1] * b[2] - a[2] * b[1],
            a[2] * b[0] - a[0] * b[2],
            a[0] * b[1] - a[1] * b[0])


def _tc_loss_body(gat_ref, pix_ref, intr_ref, out_ref, acc):
    pid = pl.program_id(0)

    @pl.when(pid == 0)
    def _():
        acc[0] = 0.0
        acc[1] = 0.0

    ir = intr_ref[...]
    rf = 1.0 / ir[:, 0:1]
    u0 = ir[:, 2:3]
    v0 = ir[:, 5:6]

    gat = gat_ref[...]
    rc = pix_ref[...]
    rowf = jnp.floor(rc * (1.0 / 512.0))
    colf = rc - rowf * 512.0

    gt, pr = [], []
    for k in range(3):
        cu = (colf[k] - u0) * rf
        cv = (rowf[k] - v0) * rf
        zt = gat[0, k]
        zp = gat[1, k]
        gt.append((cu * zt, cv * zt, zt))
        pr.append((cu * zp, cv * zp, zp))

    ds = (_sub3(gt[1], gt[0]), _sub3(gt[2], gt[0]), _sub3(gt[2], gt[1]))
    e = [[None] * 3 for _ in range(3)]
    for a in range(3):
        for b in range(a, 3):
            e[a][b] = _dot3(ds[a], ds[b])
            e[b][a] = e[a][b]
    qn = [jnp.sqrt(e[a][a]) for a in range(3)]

    hit = {}
    for a in range(3):
        for b in range(a, 3):
            thr = DELTA_COS * (qn[a] * qn[b] + EPS)
            hit[(a, b)] = (jnp.abs(e[a][b]) > thr).astype(jnp.int32)
    cnt = (hit[(0, 0)] + hit[(1, 1)] + hit[(2, 2)]
           + 2 * (hit[(0, 1)] + hit[(0, 2)] + hit[(1, 2)]))
    mask_cos = cnt > 3
    mask_pad = (gt[0][2] > DELTA_Z) & (gt[1][2] > DELTA_Z) & (gt[2][2] > DELTA_Z)
    mxyz = []
    for coord in range(3):
        mxyz.append((jnp.abs(ds[0][coord]) < DXYZ)
                    | (jnp.abs(ds[1][coord]) < DXYZ)
                    | (jnp.abs(ds[2][coord]) < DXYZ))
    keep = mask_pad & ~((mxyz[0] & mxyz[1] & mxyz[2]) | mask_cos)

    zc = [pr[k][2] == 0.0 for k in range(3)]
    prq = []
    for k in range(3):
        prq.append((jnp.where(zc[0], 1e-4, pr[k][0]),
                    jnp.where(zc[1], 1e-4, pr[k][1]),
                    jnp.where(zc[2], 1e-4, pr[k][2])))

    pp12 = _sub3(prq[1], prq[0])
    pp13 = _sub3(prq[2], prq[0])
    gn = _cross3(ds[0], ds[1])
    pn = _cross3(pp12, pp13)
    gnn = jnp.sqrt(_dot3(gn, gn))
    pnn = jnp.sqrt(_dot3(pn, pn))
    ign = 1.0 / (gnn + (gnn == 0.0).astype(jnp.float32) * EPS)
    ipn = 1.0 / (pnn + (pnn == 0.0).astype(jnp.float32) * EPS)
    lm = (jnp.abs(gn[0] * ign - pn[0] * ipn)
          + jnp.abs(gn[1] * ign - pn[1] * ipn)
          + jnp.abs(gn[2] * ign - pn[2] * ipn))

    gidx = pid * CHUNK + lax.broadcasted_iota(jnp.int32, (B, CHUNK), 1)
    w = (keep & (gidx < G)).astype(jnp.float32)
    acc[0] += jnp.sum(lm * w)
    acc[1] += jnp.sum(w)

    @pl.when(pid == NCHUNK - 1)
    def _():
        out_ref[0, 0] = acc[0] / (acc[1] * 3.0 + EPS)


_tc_loss = pl.pallas_call(
    _tc_loss_body,
    grid=(NCHUNK,),
    in_specs=[
        pl.BlockSpec((2, 3, B, CHUNK), lambda i: (0, 0, 0, i)),
        pl.BlockSpec((3, B, CHUNK), lambda i: (0, 0, i)),
        pl.BlockSpec((B, 9), lambda i: (0, 0)),
    ],
    out_specs=pl.BlockSpec((1, 1), lambda i: (0, 0), memory_space=pltpu.SMEM),
    out_shape=jax.ShapeDtypeStruct((1, 1), jnp.float32),
    scratch_shapes=[pltpu.SMEM((2,), jnp.float32)],
)


def kernel(prediction, target, mask, intrinsic):
    targ_flat = target.reshape(B * HW)
    pred_flat = prediction.reshape(B * HW)
    gat = _get_sc_gather()(targ_flat, pred_flat, _SCIDX)
    out = _tc_loss(gat, _RC, intrinsic.reshape(B, 9))
    return out.reshape(())
